# Initial kernel scaffold; baseline (speedup 1.0000x reference)
import jax
import jax.numpy as jnp
from jax.experimental import pallas as pl


def kernel(scene_tokens, scene_xyz, learned_queries, learned_xyz):
    raise NotImplementedError("write your pallas kernel here")


# speedup vs baseline: 3.4694x; 3.4694x over previous
import jax
import jax.numpy as jnp
import numpy as np
from jax import lax
from jax.experimental import pallas as pl
from jax.experimental.pallas import tpu as pltpu
from jax.experimental.pallas import tpu_sc as plsc

HIDDEN_DIM = 256
NUM_QUERIES = 4096
NUM_LEARNED = NUM_QUERIES // 4
NUM_SCENE = NUM_QUERIES - NUM_LEARNED
V = 100000

NC, NS = 2, 16
NW = NC * NS
BPW = NUM_SCENE // NW
LPW = NUM_LEARNED // NW
XW = 3 * BPW

_CONST_CACHE = {}


def _rotl32(x, d):
    d = np.uint32(d)
    return (x << d) | (x >> np.uint32(32 - d))


def _threefry2x32(k1, k2, x1, x2):
    rot_a = (13, 15, 26, 6)
    rot_b = (17, 29, 16, 24)
    ks = [np.uint32(k1), np.uint32(k2),
          np.uint32(k1) ^ np.uint32(k2) ^ np.uint32(0x1BD11BDA)]
    a = (x1 + ks[0]).astype(np.uint32)
    b = (x2 + ks[1]).astype(np.uint32)

    def rounds(a, b, rots):
        for r in rots:
            a = (a + b).astype(np.uint32)
            b = a ^ _rotl32(b, r)
        return a, b

    for i, (rots, ka, kb) in enumerate(
            [(rot_a, 1, 2), (rot_b, 2, 0), (rot_a, 0, 1),
             (rot_b, 1, 2), (rot_a, 2, 0)]):
        a, b = rounds(a, b, rots)
        a = (a + ks[ka]).astype(np.uint32)
        b = (b + ks[kb] + np.uint32(i + 1)).astype(np.uint32)
    return a, b


def _np_permutation_key1(n):
    key = (np.uint32(0), np.uint32(1))
    x = np.arange(n, dtype=np.int32)
    uint32max = np.iinfo(np.uint32).max
    num_rounds = int(np.ceil(3 * np.log(max(1, n)) / np.log(uint32max)))
    for _ in range(num_rounds):
        b1, b2 = _threefry2x32(key[0], key[1],
                               np.zeros(2, np.uint32),
                               np.arange(2, dtype=np.uint32))
        key, subkey = (b1[0], b2[0]), (b1[1], b2[1])
        s1, s2 = _threefry2x32(subkey[0], subkey[1],
                               np.zeros(n, np.uint32),
                               np.arange(n, dtype=np.uint32))
        x = x[np.argsort(s1 ^ s2, kind="stable")]
    return x


def _scene_indices():
    if "idx" not in _CONST_CACHE:
        idx = _np_permutation_key1(V)[:NUM_SCENE].astype(np.int32)
        idx3 = (3 * idx[:, None] + np.arange(3, dtype=np.int32)).reshape(-1)
        _CONST_CACHE["idx"] = idx
        _CONST_CACHE["idx3"] = idx3
    return _CONST_CACHE["idx"], _CONST_CACHE["idx3"]


def _sc_gather_concat(tok, xyz_flat, lq, lx_flat, idx, idx3):
    mesh = plsc.VectorSubcoreMesh(core_axis_name="c", subcore_axis_name="s")

    def body(tok_h, xyzf_h, lq_h, lxf_h, idx_h, idx3_h, out_f, out_x,
             idx_v, rows_v, lq_v, idx3_v, xyz_v, lx_v, sem):
        wid = lax.axis_index("s") * NC + lax.axis_index("c")
        base = wid * BPW
        pltpu.sync_copy(idx_h.at[pl.ds(base, BPW)], idx_v)
        pltpu.async_copy(tok_h.at[idx_v], rows_v, sem).wait()
        pltpu.sync_copy(rows_v, out_f.at[pl.ds(base, BPW)])
        lbase = wid * LPW
        pltpu.sync_copy(lq_h.at[pl.ds(lbase, LPW)], lq_v)
        pltpu.sync_copy(lq_v, out_f.at[pl.ds(NUM_SCENE + lbase, LPW)])
        for t in range(3):
            pltpu.sync_copy(idx3_h.at[pl.ds(wid * XW + t * BPW, BPW)], idx3_v)
            pltpu.async_copy(xyzf_h.at[idx3_v], xyz_v.at[pl.ds(t * BPW, BPW)],
                             sem).wait()
        pltpu.sync_copy(xyz_v, out_x.at[pl.ds(wid * XW, XW)])
        pltpu.sync_copy(lxf_h.at[pl.ds(wid * 3 * LPW, 3 * LPW)], lx_v)
        pltpu.sync_copy(lx_v, out_x.at[pl.ds(3 * NUM_SCENE + wid * 3 * LPW,
                                             3 * LPW)])

    run = pl.kernel(
        body,
        mesh=mesh,
        out_type=(
            jax.ShapeDtypeStruct((NUM_QUERIES, HIDDEN_DIM), jnp.float32),
            jax.ShapeDtypeStruct((3 * NUM_QUERIES,), jnp.float32),
        ),
        scratch_types=[
            pltpu.VMEM((BPW,), jnp.int32),
            pltpu.VMEM((BPW, HIDDEN_DIM), jnp.float32),
            pltpu.VMEM((LPW, HIDDEN_DIM), jnp.float32),
            pltpu.VMEM((BPW,), jnp.int32),
            pltpu.VMEM((XW,), jnp.float32),
            pltpu.VMEM((3 * LPW,), jnp.float32),
            pltpu.SemaphoreType.DMA,
        ],
    )
    return run(tok, xyz_flat, lq, lx_flat, idx, idx3)


def kernel(scene_tokens, scene_xyz, learned_queries, learned_xyz):
    idx, idx3 = _scene_indices()
    q_feat, q_xyz_flat = _sc_gather_concat(
        scene_tokens,
        scene_xyz.reshape(-1),
        learned_queries,
        learned_xyz.reshape(-1),
        jnp.asarray(idx),
        jnp.asarray(idx3),
    )
    return (q_feat, q_xyz_flat.reshape(NUM_QUERIES, 3))


# speedup vs baseline: 3.6392x; 1.0489x over previous
import jax
import jax.numpy as jnp
import numpy as np
from jax import lax
from jax.experimental import pallas as pl
from jax.experimental.pallas import tpu as pltpu
from jax.experimental.pallas import tpu_sc as plsc

HIDDEN_DIM = 256
NUM_QUERIES = 4096
NUM_LEARNED = NUM_QUERIES // 4
NUM_SCENE = NUM_QUERIES - NUM_LEARNED
V = 100000

NC, NS = 2, 16
NW = NC * NS
BPW = NUM_SCENE // NW
LPW = NUM_LEARNED // NW
XW = 3 * BPW

_CONST_CACHE = {}


def _rotl32(x, d):
    d = np.uint32(d)
    return (x << d) | (x >> np.uint32(32 - d))


def _threefry2x32(k1, k2, x1, x2):
    rot_a = (13, 15, 26, 6)
    rot_b = (17, 29, 16, 24)
    ks = [np.uint32(k1), np.uint32(k2),
          np.uint32(k1) ^ np.uint32(k2) ^ np.uint32(0x1BD11BDA)]
    a = (x1 + ks[0]).astype(np.uint32)
    b = (x2 + ks[1]).astype(np.uint32)

    def rounds(a, b, rots):
        for r in rots:
            a = (a + b).astype(np.uint32)
            b = a ^ _rotl32(b, r)
        return a, b

    for i, (rots, ka, kb) in enumerate(
            [(rot_a, 1, 2), (rot_b, 2, 0), (rot_a, 0, 1),
             (rot_b, 1, 2), (rot_a, 2, 0)]):
        a, b = rounds(a, b, rots)
        a = (a + ks[ka]).astype(np.uint32)
        b = (b + ks[kb] + np.uint32(i + 1)).astype(np.uint32)
    return a, b


def _np_permutation_key1(n):
    key = (np.uint32(0), np.uint32(1))
    x = np.arange(n, dtype=np.int32)
    uint32max = np.iinfo(np.uint32).max
    num_rounds = int(np.ceil(3 * np.log(max(1, n)) / np.log(uint32max)))
    for _ in range(num_rounds):
        b1, b2 = _threefry2x32(key[0], key[1],
                               np.zeros(2, np.uint32),
                               np.arange(2, dtype=np.uint32))
        key, subkey = (b1[0], b2[0]), (b1[1], b2[1])
        s1, s2 = _threefry2x32(subkey[0], subkey[1],
                               np.zeros(n, np.uint32),
                               np.arange(n, dtype=np.uint32))
        x = x[np.argsort(s1 ^ s2, kind="stable")]
    return x


def _scene_indices():
    if "idx" not in _CONST_CACHE:
        idx = _np_permutation_key1(V)[:NUM_SCENE].astype(np.int32)
        idx3 = (3 * idx[:, None] + np.arange(3, dtype=np.int32)).reshape(-1)
        _CONST_CACHE["idx"] = idx
        _CONST_CACHE["idx3"] = idx3
    return _CONST_CACHE["idx"], _CONST_CACHE["idx3"]


def _sc_gather_concat(tok, xyz_flat, lq, lx_flat, idx, idx3):
    mesh = plsc.VectorSubcoreMesh(core_axis_name="c", subcore_axis_name="s")

    def body(tok_h, xyzf_h, lq_h, lxf_h, idx_h, idx3_h, out_f, out_x,
             idx_v, rows_v, lq_v, idx3_v, xyz_v, lx_v,
             sem_i, sem_l, sem_g, sem_w):
        wid = lax.axis_index("s") * NC + lax.axis_index("c")
        base = wid * BPW
        lbase = wid * LPW
        c_idx = pltpu.async_copy(idx_h.at[pl.ds(base, BPW)], idx_v, sem_i)
        c_idx3 = pltpu.async_copy(idx3_h.at[pl.ds(wid * XW, XW)], idx3_v,
                                  sem_i)
        c_lq = pltpu.async_copy(lq_h.at[pl.ds(lbase, LPW)], lq_v, sem_l)
        c_lx = pltpu.async_copy(lxf_h.at[pl.ds(wid * 3 * LPW, 3 * LPW)], lx_v,
                                sem_l)
        c_idx.wait()
        c_idx3.wait()
        g_tok = pltpu.async_copy(tok_h.at[idx_v], rows_v, sem_g)
        g_xyz = [
            pltpu.async_copy(xyzf_h.at[idx3_v.at[pl.ds(t * BPW, BPW)]],
                             xyz_v.at[pl.ds(t * BPW, BPW)], sem_g)
            for t in range(3)
        ]
        c_lq.wait()
        c_lx.wait()
        w_lq = pltpu.async_copy(lq_v, out_f.at[pl.ds(NUM_SCENE + lbase, LPW)],
                                sem_w)
        w_lx = pltpu.async_copy(lx_v,
                                out_x.at[pl.ds(3 * NUM_SCENE + wid * 3 * LPW,
                                               3 * LPW)], sem_w)
        g_tok.wait()
        for g in g_xyz:
            g.wait()
        w_tok = pltpu.async_copy(rows_v, out_f.at[pl.ds(base, BPW)], sem_w)
        w_xyz = pltpu.async_copy(xyz_v, out_x.at[pl.ds(wid * XW, XW)], sem_w)
        w_lq.wait()
        w_lx.wait()
        w_tok.wait()
        w_xyz.wait()

    run = pl.kernel(
        body,
        mesh=mesh,
        out_type=(
            jax.ShapeDtypeStruct((NUM_QUERIES, HIDDEN_DIM), jnp.float32),
            jax.ShapeDtypeStruct((3 * NUM_QUERIES,), jnp.float32),
        ),
        scratch_types=[
            pltpu.VMEM((BPW,), jnp.int32),
            pltpu.VMEM((BPW, HIDDEN_DIM), jnp.float32),
            pltpu.VMEM((LPW, HIDDEN_DIM), jnp.float32),
            pltpu.VMEM((XW,), jnp.int32),
            pltpu.VMEM((XW,), jnp.float32),
            pltpu.VMEM((3 * LPW,), jnp.float32),
            pltpu.SemaphoreType.DMA,
            pltpu.SemaphoreType.DMA,
            pltpu.SemaphoreType.DMA,
            pltpu.SemaphoreType.DMA,
        ],
    )
    return run(tok, xyz_flat, lq, lx_flat, idx, idx3)


def kernel(scene_tokens, scene_xyz, learned_queries, learned_xyz):
    idx, idx3 = _scene_indices()
    q_feat, q_xyz_flat = _sc_gather_concat(
        scene_tokens,
        scene_xyz.reshape(-1),
        learned_queries,
        learned_xyz.reshape(-1),
        jnp.asarray(idx),
        jnp.asarray(idx3),
    )
    return (q_feat, q_xyz_flat.reshape(NUM_QUERIES, 3))


# speedup vs baseline: 6.2779x; 1.7251x over previous
import jax
import jax.numpy as jnp
import numpy as np
from jax import lax
from jax.experimental import pallas as pl
from jax.experimental.pallas import tpu as pltpu
from jax.experimental.pallas import tpu_sc as plsc

HIDDEN_DIM = 256
NUM_QUERIES = 4096
NUM_LEARNED = NUM_QUERIES // 4
NUM_SCENE = NUM_QUERIES - NUM_LEARNED
V = 100000

NC, NS = 2, 16
NW = NC * NS
BPW = NUM_SCENE // NW
LPW = NUM_LEARNED // NW

_CONST_CACHE = {}


def _rotl32(x, d):
    d = np.uint32(d)
    return (x << d) | (x >> np.uint32(32 - d))


def _threefry2x32(k1, k2, x1, x2):
    rot_a = (13, 15, 26, 6)
    rot_b = (17, 29, 16, 24)
    ks = [np.uint32(k1), np.uint32(k2),
          np.uint32(k1) ^ np.uint32(k2) ^ np.uint32(0x1BD11BDA)]
    a = (x1 + ks[0]).astype(np.uint32)
    b = (x2 + ks[1]).astype(np.uint32)

    def rounds(a, b, rots):
        for r in rots:
            a = (a + b).astype(np.uint32)
            b = a ^ _rotl32(b, r)
        return a, b

    for i, (rots, ka, kb) in enumerate(
            [(rot_a, 1, 2), (rot_b, 2, 0), (rot_a, 0, 1),
             (rot_b, 1, 2), (rot_a, 2, 0)]):
        a, b = rounds(a, b, rots)
        a = (a + ks[ka]).astype(np.uint32)
        b = (b + ks[kb] + np.uint32(i + 1)).astype(np.uint32)
    return a, b


def _np_permutation_key1(n):
    key = (np.uint32(0), np.uint32(1))
    x = np.arange(n, dtype=np.int32)
    uint32max = np.iinfo(np.uint32).max
    num_rounds = int(np.ceil(3 * np.log(max(1, n)) / np.log(uint32max)))
    for _ in range(num_rounds):
        b1, b2 = _threefry2x32(key[0], key[1],
                               np.zeros(2, np.uint32),
                               np.arange(2, dtype=np.uint32))
        key, subkey = (b1[0], b2[0]), (b1[1], b2[1])
        s1, s2 = _threefry2x32(subkey[0], subkey[1],
                               np.zeros(n, np.uint32),
                               np.arange(n, dtype=np.uint32))
        x = x[np.argsort(s1 ^ s2, kind="stable")]
    return x


def _scene_indices():
    if "idx" not in _CONST_CACHE:
        _CONST_CACHE["idx"] = _np_permutation_key1(V)[:NUM_SCENE].astype(
            np.int32)
    return _CONST_CACHE["idx"]


def _sc_gather_concat(tok, xyz, lq, lx, idx):
    mesh = plsc.VectorSubcoreMesh(core_axis_name="c", subcore_axis_name="s")

    def body(tok_h, xyz_h, lq_h, lx_h, idx_h, out_f, out_x,
             idx_v, rows_v, lq_v, xyz_v, lx_v,
             sem_i, sem_l, sem_g, sem_w, sem_x):
        wid = lax.axis_index("s") * NC + lax.axis_index("c")
        base = wid * BPW
        lbase = wid * LPW
        c_idx = pltpu.async_copy(idx_h.at[pl.ds(base, BPW)], idx_v, sem_i)
        c_lq = pltpu.async_copy(lq_h.at[pl.ds(lbase, LPW)], lq_v, sem_l)
        c_lx = pltpu.async_copy(lx_h.at[pl.ds(lbase, LPW)], lx_v, sem_l)
        c_idx.wait()
        g_tok = pltpu.async_copy(tok_h.at[idx_v], rows_v, sem_g)

        for k in range(BPW // 16):
            vec = idx_v[pl.ds(16 * k, 16)]
            for j in range(16):
                row = vec[j]
                pltpu.async_copy(xyz_h.at[pl.ds(row, 1)],
                                 xyz_v.at[pl.ds(16 * k + j, 1)], sem_x)
        c_lq.wait()
        c_lx.wait()
        w_lq = pltpu.async_copy(lq_v, out_f.at[pl.ds(NUM_SCENE + lbase, LPW)],
                                sem_w)
        w_lx = pltpu.async_copy(lx_v, out_x.at[pl.ds(NUM_SCENE + lbase, LPW)],
                                sem_w)
        g_tok.wait()
        pltpu.make_async_copy(xyz_h.at[pl.ds(0, BPW)], xyz_v, sem_x).wait()
        w_tok = pltpu.async_copy(rows_v, out_f.at[pl.ds(base, BPW)], sem_w)
        w_xyz = pltpu.async_copy(xyz_v, out_x.at[pl.ds(base, BPW)], sem_w)
        w_lq.wait()
        w_lx.wait()
        w_tok.wait()
        w_xyz.wait()

    run = pl.kernel(
        body,
        mesh=mesh,
        out_type=(
            jax.ShapeDtypeStruct((NUM_QUERIES, HIDDEN_DIM), jnp.float32),
            jax.ShapeDtypeStruct((NUM_QUERIES, 3), jnp.float32),
        ),
        scratch_types=[
            pltpu.VMEM((BPW,), jnp.int32),
            pltpu.VMEM((BPW, HIDDEN_DIM), jnp.float32),
            pltpu.VMEM((LPW, HIDDEN_DIM), jnp.float32),
            pltpu.VMEM((BPW, 3), jnp.float32),
            pltpu.VMEM((LPW, 3), jnp.float32),
            pltpu.SemaphoreType.DMA,
            pltpu.SemaphoreType.DMA,
            pltpu.SemaphoreType.DMA,
            pltpu.SemaphoreType.DMA,
            pltpu.SemaphoreType.DMA,
        ],
    )
    return run(tok, xyz, lq, lx, idx)


def kernel(scene_tokens, scene_xyz, learned_queries, learned_xyz):
    idx = _scene_indices()
    return _sc_gather_concat(scene_tokens, scene_xyz, learned_queries,
                             learned_xyz, jnp.asarray(idx))
